# baseline (speedup 1.0000x reference)
import functools

import jax
import jax.numpy as jnp
from jax import lax
from jax.experimental import pallas as pl
from jax.experimental.pallas import tpu as pltpu
from jax.experimental.pallas import tpu_sc as plsc

N_ROW_BLK = 1000
N_NODES = 10000
N_EDGES = 320000
D = 128
ROWW = 144
NC = 2
NS = 16
NW = NC * NS
CHUNK = 64
IDX_ROWS = N_EDGES // CHUNK
NPAD = 10240
ROWS_PER_NODE_SLICE = NPAD // NS


def _enc_body(x_ref, pos_ref, W0a_ref, W0b_ref, b0_ref, W1_ref, b1_ref,
              W2_ref, b2_ref, Wl_ref, bl_ref, Wr_ref, br_ref,
              xl_ref, xr_ref):
    x = x_ref[...]
    pos = pos_ref[...]
    h = x @ W0a_ref[...] + pos @ W0b_ref[...] + b0_ref[...]
    h = jnp.maximum(h, 0.0)
    h = jnp.maximum(h @ W1_ref[...] + b1_ref[...], 0.0)
    h = h @ W2_ref[...] + b2_ref[...]
    xl_ref[...] = h @ Wl_ref[...] + bl_ref[...]
    xr_ref[...] = h @ Wr_ref[...] + br_ref[...]


def _encoder(x, pos, W0, b0, W1, b1, W2, b2, Wl, bl, Wr, br):
    n = x.shape[0]
    grid = (n // N_ROW_BLK,)
    row = lambda i: (i, 0)
    rep = lambda i: (0, 0)
    out_shape = [jax.ShapeDtypeStruct((n, 128), jnp.float32)] * 2
    return pl.pallas_call(
        _enc_body,
        grid=grid,
        in_specs=[
            pl.BlockSpec((N_ROW_BLK, 128), row),
            pl.BlockSpec((N_ROW_BLK, 2), row),
            pl.BlockSpec((128, 512), rep),
            pl.BlockSpec((2, 512), rep),
            pl.BlockSpec((512,), lambda i: (0,)),
            pl.BlockSpec((512, 256), rep),
            pl.BlockSpec((256,), lambda i: (0,)),
            pl.BlockSpec((256, 128), rep),
            pl.BlockSpec((128,), lambda i: (0,)),
            pl.BlockSpec((128, 128), rep),
            pl.BlockSpec((128,), lambda i: (0,)),
            pl.BlockSpec((128, 128), rep),
            pl.BlockSpec((128,), lambda i: (0,)),
        ],
        out_specs=[pl.BlockSpec((N_ROW_BLK, 128), row)] * 2,
        out_shape=out_shape,
    )(x, pos, W0[:128], W0[128:], b0, W1, b1, W2, b2, Wl, bl, Wr, br)


def _edge_body(xl_hbm, xr_hbm, src_hbm, dst_hbm, attr_hbm, vecs_hbm,
               out_hbm, acc, srci, dsti, attrv, A, B, W, vecs, zbuf,
               gsem_a, gsem_b):
    cid = lax.axis_index("c")
    sid = lax.axis_index("s")
    w = cid * NS + sid
    start = (w * IDX_ROWS) // NW
    end = ((w + 1) * IDX_ROWS) // NW

    zeros16 = jnp.zeros((16,), jnp.float32)
    rows16 = lax.iota(jnp.int32, 16)

    def _zrow(i, _):
        def _zcol(j, _):
            zbuf[i, pl.ds(j * 16, 16)] = zeros16
            return 0
        return lax.fori_loop(0, ROWW // 16, _zcol, 0)
    lax.fori_loop(0, 16, _zrow, 0)
    nbase = sid * ROWS_PER_NODE_SLICE

    def _zcp(k, _):
        pltpu.sync_copy(zbuf, acc.at[pl.ds(nbase + k * 16, 16)])
        return 0
    lax.fori_loop(0, ROWS_PER_NODE_SLICE // 16, _zcp, 0)

    pltpu.sync_copy(vecs_hbm, vecs)

    plsc.subcore_barrier()

    def _chunk(r, _):
        pltpu.sync_copy(src_hbm.at[r], srci.at[0])
        pltpu.sync_copy(dst_hbm.at[r], dsti.at[0])
        pltpu.sync_copy(attr_hbm.at[r], attrv)
        ga = pltpu.async_copy(xl_hbm.at[srci.at[0]], A, gsem_a)
        gb = pltpu.async_copy(xr_hbm.at[dsti.at[0]], B, gsem_b)
        ga.wait()
        gb.wait()

        attr_g = [attrv[pl.ds(g * 16, 16)] for g in range(CHUNK // 16)]

        def _fbody(f, acc8):
            fs = jnp.full((16,), f, jnp.int32)
            wef = plsc.load_gather(vecs, [jnp.zeros((16,), jnp.int32), fs])
            attf = plsc.load_gather(vecs, [jnp.ones((16,), jnp.int32), fs])
            out = []
            for g in range(CHUNK // 16):
                rg = rows16 + (g * 16)
                xa = plsc.load_gather(A, [rg, fs])
                xb = plsc.load_gather(B, [rg, fs])
                m = xa + xb + attr_g[g] * wef
                m = jnp.where(m > 0, m, 0.2 * m)
                out.append(acc8[g] + m * attf)
            return out
        logits = lax.fori_loop(0, D, _fbody, [zeros16] * (CHUNK // 16))
        ex_g = [jnp.exp(lg) for lg in logits]

        c128 = jnp.full((16,), D, jnp.int32)
        for g in range(CHUNK // 16):
            rg = rows16 + (g * 16)
            plsc.store_scatter(W, [rg, c128], ex_g[g])

        def _fbody2(f, _):
            fs = jnp.full((16,), f, jnp.int32)
            for g in range(CHUNK // 16):
                rg = rows16 + (g * 16)
                xa = plsc.load_gather(A, [rg, fs])
                plsc.store_scatter(W, [rg, fs], xa * ex_g[g])
            return 0
        lax.fori_loop(0, D, _fbody2, 0)

        pltpu.sync_copy(W, acc.at[dsti.at[0]], add=True)
        return 0

    lax.fori_loop(start, end, _chunk, 0)

    plsc.subcore_barrier()

    def _ocp(k, _):
        pltpu.sync_copy(acc.at[pl.ds(nbase + k * 128, 128)],
                        out_hbm.at[cid, pl.ds(nbase + k * 128, 128)])
        return 0
    lax.fori_loop(0, ROWS_PER_NODE_SLICE // 128, _ocp, 0)


def _edge_phase(xl, xr, src2d, dst2d, attr2d, vecs):
    mesh = plsc.VectorSubcoreMesh(core_axis_name="c", subcore_axis_name="s")
    f = pl.kernel(
        _edge_body,
        out_type=jax.ShapeDtypeStruct((NC, NPAD, ROWW), jnp.float32),
        mesh=mesh,
        compiler_params=pltpu.CompilerParams(
            needs_layout_passes=False, use_tc_tiling_on_sc=False),
        scratch_types=[
            pltpu.VMEM_SHARED((NPAD, ROWW), jnp.float32),
            pltpu.VMEM((1, CHUNK), jnp.int32),
            pltpu.VMEM((1, CHUNK), jnp.int32),
            pltpu.VMEM((CHUNK,), jnp.float32),
            pltpu.VMEM((CHUNK, D), jnp.float32),
            pltpu.VMEM((CHUNK, D), jnp.float32),
            pltpu.VMEM((CHUNK, ROWW), jnp.float32),
            pltpu.VMEM((2, D), jnp.float32),
            pltpu.VMEM((16, ROWW), jnp.float32),
            pltpu.SemaphoreType.DMA,
            pltpu.SemaphoreType.DMA,
        ],
    )
    return f(xl, xr, src2d, dst2d, attr2d, vecs)


def _dec_body(u_ref, bias_ref, Wd0_ref, bd0_ref, Wd1_ref, bd1_ref, out_ref):
    num = u_ref[0, :, :D] + u_ref[1, :, :D]
    den = u_ref[0, :, D] + u_ref[1, :, D]
    agg = num / (den[:, None] + 1e-16) + bias_ref[...]
    z = jnp.maximum(agg, 0.0)
    d = jnp.maximum(z @ Wd0_ref[...] + bd0_ref[...], 0.0)
    logits = d @ Wd1_ref[...] + bd1_ref[...]
    out_ref[...] = jax.nn.softmax(logits, axis=-1)


def _decoder(u, bias_g, Wd0, bd0, Wd1, bd1):
    n = N_NODES
    grid = (n // N_ROW_BLK,)
    rep = lambda i: (0, 0)
    return pl.pallas_call(
        _dec_body,
        grid=grid,
        in_specs=[
            pl.BlockSpec((NC, N_ROW_BLK, ROWW), lambda i: (0, i, 0)),
            pl.BlockSpec((128,), lambda i: (0,)),
            pl.BlockSpec((128, 64), rep),
            pl.BlockSpec((64,), lambda i: (0,)),
            pl.BlockSpec((64, 30), rep),
            pl.BlockSpec((30,), lambda i: (0,)),
        ],
        out_specs=pl.BlockSpec((N_ROW_BLK, 30), lambda i: (i, 0)),
        out_shape=jax.ShapeDtypeStruct((n, 30), jnp.float32),
    )(u, bias_g, Wd0, bd0, Wd1, bd1)


def kernel(x, edge_index, edge_attr, pos, W0, b0, W1, b1, W2, b2, Wl, bl,
           Wr, br, We, att, bias_g, Wd0, bd0, Wd1, bd1):
    xl, xr = _encoder(x, pos, W0, b0, W1, b1, W2, b2, Wl, bl, Wr, br)
    src2d = edge_index[0].reshape(IDX_ROWS, CHUNK)
    dst2d = edge_index[1].reshape(IDX_ROWS, CHUNK)
    attr2d = edge_attr.reshape(IDX_ROWS, CHUNK)
    vecs = jnp.stack([We[0], att])
    u = _edge_phase(xl, xr, src2d, dst2d, attr2d, vecs)
    return _decoder(u, bias_g, Wd0, bd0, Wd1, bd1)
